# baseline (device time: 80554 ns/iter reference)
import jax
import jax.numpy as jnp
from jax import lax
from jax.experimental import pallas as pl
from jax.experimental.pallas import tpu as pltpu

B, H, D, BS = 16, 16, 64, 16
NB = 128
P_LOCAL = 128
NKEY = P_LOCAL * BS
NREP = 8
P_CHUNK = P_LOCAL // NREP
KEY_CHUNK = P_CHUNK * BS
NSTAGE = 4
NEG = -1e30


def kernel(Q, K, V, bt, lens):
    qh = Q[:, 0].transpose(1, 0, 2)
    kh = K.reshape(NKEY, H, D).transpose(1, 2, 0)
    vh = V.reshape(NKEY, H, D).transpose(1, 0, 2)
    bt3 = bt.reshape(B, NB, 1)
    lens3 = lens.reshape(B, 1, 1)

    def body(q_ref, k_ref, v_ref, bt_ref, lens_ref, out_ref,
             acc_comm, stats_comm, acc_send, acc_recv, st_send, st_recv):
        my_x = lax.axis_index("x")
        my_y = lax.axis_index("y")
        my_z = lax.axis_index("z")
        partners = [
            (1 - my_x, my_y, my_z),
            (my_x, 1 - my_y, my_z),
            (my_x, my_y, my_z ^ 1),
            (my_x, my_y, my_z ^ 2),
        ]

        barrier = pltpu.get_barrier_semaphore()
        for p in partners:
            pl.semaphore_signal(barrier, inc=1, device_id=p,
                                device_id_type=pl.DeviceIdType.MESH)
        pl.semaphore_wait(barrier, NSTAGE)

        r = my_y * (2 * 2) + my_z
        page_base = r * P_CHUNK

        btv = bt_ref[...]
        lensv = lens_ref[...]
        slot_iota = lax.broadcasted_iota(jnp.int32, (B, NB, P_CHUNK), 1)
        page_iota = lax.broadcasted_iota(jnp.int32, (B, NB, P_CHUNK), 2)
        win_bt = jnp.broadcast_to(btv - (my_x * P_LOCAL + page_base),
                                  (B, NB, P_CHUNK))
        slot_ok = slot_iota < jnp.broadcast_to(lensv, (B, NB, P_CHUNK))
        hit = (win_bt == page_iota) & slot_ok
        counts = jnp.sum(hit.astype(jnp.float32), axis=1)

        row = lax.broadcasted_iota(jnp.int32, (P_CHUNK, KEY_CHUNK), 0)
        col = lax.broadcasted_iota(jnp.int32, (P_CHUNK, KEY_CHUNK), 1)
        expand = ((col >= row * BS) & (col < row * BS + BS)).astype(jnp.float32)
        w = jnp.dot(counts, expand,
                    preferred_element_type=jnp.float32)
        sel = w > 0.0

        scale = D ** -0.5
        key_base = r * KEY_CHUNK
        ms, ls, accs = [], [], []
        for h in range(H):
            kk = k_ref[h, :, pl.ds(key_base, KEY_CHUNK)]
            s = jnp.dot(q_ref[h], kk,
                        preferred_element_type=jnp.float32) * scale
            m = jnp.max(jnp.where(sel, s, NEG), axis=-1, keepdims=True)
            e = w * jnp.exp(jnp.where(sel, s - m, NEG))
            l = jnp.sum(e, axis=-1, keepdims=True)
            acc = jnp.dot(e, v_ref[h, pl.ds(key_base, KEY_CHUNK), :],
                          preferred_element_type=jnp.float32)
            ms.append(m)
            ls.append(l)
            accs.append(acc)

        for st in range(NSTAGE):
            for h in range(H):
                acc_comm[st, 0, h] = accs[h]
                stats_comm[st, 0, 0, h] = ms[h]
                stats_comm[st, 0, 1, h] = ls[h]
            copies = [
                pltpu.make_async_remote_copy(
                    src_ref=ref.at[st, 0], dst_ref=ref.at[st, 1],
                    send_sem=ssem.at[st], recv_sem=rsem.at[st],
                    device_id=partners[st],
                    device_id_type=pl.DeviceIdType.MESH,
                )
                for ref, ssem, rsem in (
                    (acc_comm, acc_send, acc_recv),
                    (stats_comm, st_send, st_recv),
                )
            ]
            for c in copies:
                c.start()
            for c in copies:
                c.wait()
            for h in range(H):
                m2 = stats_comm[st, 1, 0, h]
                l2 = stats_comm[st, 1, 1, h]
                acc2 = acc_comm[st, 1, h]
                mm = jnp.maximum(ms[h], m2)
                a1 = jnp.exp(ms[h] - mm)
                a2 = jnp.exp(m2 - mm)
                ms[h] = mm
                ls[h] = ls[h] * a1 + l2 * a2
                accs[h] = accs[h] * a1 + acc2 * a2

        for h in range(H):
            out_ref[h] = accs[h] / ls[h]

    out = pl.pallas_call(
        body,
        out_shape=jax.ShapeDtypeStruct((H, B, D), jnp.float32),
        in_specs=[pl.BlockSpec(memory_space=pltpu.VMEM)] * 5,
        out_specs=pl.BlockSpec(memory_space=pltpu.VMEM),
        scratch_shapes=[
            pltpu.VMEM((NSTAGE, 2, H, B, D), jnp.float32),
            pltpu.VMEM((NSTAGE, 2, 2, H, B, 1), jnp.float32),
            pltpu.SemaphoreType.DMA((NSTAGE,)),
            pltpu.SemaphoreType.DMA((NSTAGE,)),
            pltpu.SemaphoreType.DMA((NSTAGE,)),
            pltpu.SemaphoreType.DMA((NSTAGE,)),
        ],
        compiler_params=pltpu.CompilerParams(collective_id=0),
    )(qh, kh, vh, bt3, lens3)
    return out.transpose(1, 0, 2)[:, None]
